# Initial kernel scaffold; baseline (speedup 1.0000x reference)
import jax
import jax.numpy as jnp
from jax.experimental import pallas as pl


def kernel(node_positions, node_vectors, node_features, senders, receivers, We1, be1, We2, be2, Wy1, by1, Wy2, by2, Wyo, byo, Winf, binf, Wh1, bh1, Wh2, bh2, Wh3, bh3):
    raise NotImplementedError("write your pallas kernel here")


# speedup vs baseline: 15.8019x; 15.8019x over previous
import functools
import math

import jax
import jax.numpy as jnp
import numpy as np
from jax import lax
from jax.experimental import pallas as pl
from jax.experimental.pallas import tpu as pltpu
from jax.experimental.pallas import tpu_sc as plsc

_N = 10000
_E = 320000
_F = 128
_H = 64

_NC = 2
_NS = 16
_NW = _NC * _NS
_EPW = _E // _NW
_C = 80
_NCHUNK = _EPW // _C
_TROW = 80
_OROW = 80
_NPT = _N // _NS
_ZR = 25

_BN = 1000
_BE = 1000

_DEBUG_FAKE_GATHER = False
_DEBUG_FAKE_SCATTER = False


def _tables_body(feat, pos, vec, we1s, we1r, ts, tr):
    f = feat[...]
    a = jnp.dot(f, we1s[...], preferred_element_type=jnp.float32)
    b = jnp.dot(f, we1r[...], preferred_element_type=jnp.float32)
    z = jnp.zeros((f.shape[0], 2), jnp.float32)
    geo = jnp.concatenate([pos[...], z, vec[...], z], axis=1)
    ts[...] = jnp.concatenate([a, geo], axis=1)
    tr[...] = jnp.concatenate([b, geo], axis=1)


def _edge_body(g, cd, sel, wsq, be1, we2, be2, wy1, by1, wy2, by2, wyo, byo,
               winf, binf, out):
    gb = g[...]
    cdb = cd[...]
    d2 = cdb * cdb
    sq = jnp.dot(d2, sel[...], preferred_element_type=jnp.float32)
    pre1 = gb + jnp.dot(sq, wsq[...], preferred_element_type=jnp.float32) + be1[...]
    h1 = jax.nn.silu(pre1)
    m = jax.nn.silu(jnp.dot(h1, we2[...], preferred_element_type=jnp.float32) + be2[...])
    t1 = jax.nn.silu(jnp.dot(m, wy1[...], preferred_element_type=jnp.float32) + by1[...])
    t2 = jax.nn.silu(jnp.dot(t1, wy2[...], preferred_element_type=jnp.float32) + by2[...])
    phi = jnp.dot(t2, wyo[...], preferred_element_type=jnp.float32) + byo[...]
    e = jax.nn.sigmoid(jnp.dot(m, winf[...], preferred_element_type=jnp.float32) + binf[...])
    mg = m * e
    sy0 = sq[:, 2:3]
    sy1 = sq[:, 3:4]
    s0 = phi[:, 0:1] / (1.0 + jnp.sqrt(sy0))
    s1 = phi[:, 1:2] / (1.0 + jnp.sqrt(sy1))
    sh0 = cdb[:, 8:11] * s0
    sh1 = cdb[:, 11:14] * s1
    pad = jnp.zeros((gb.shape[0], _OROW - _H - 6), jnp.float32)
    out[...] = jnp.concatenate([mg, sh0, sh1, pad], axis=1)


def _node_body(p, feat, vec, wh1m, wh1f, bh1, wh2, bh2, wh3, bh3,
               fo, vo, rv):
    inv_avg = 1.0 / float(int(math.sqrt(_E)))
    acc = (p[0] + p[1]) * inv_avg
    m_i = acc[:, 0:_H]
    sh = acc[:, _H:_H + 6]
    f = feat[...]
    h = jax.nn.silu(jnp.dot(m_i, wh1m[...], preferred_element_type=jnp.float32)
                    + jnp.dot(f, wh1f[...], preferred_element_type=jnp.float32)
                    + bh1[...])
    h2 = jax.nn.silu(jnp.dot(h, wh2[...], preferred_element_type=jnp.float32) + bh2[...])
    rf = jnp.dot(h2, wh3[...], preferred_element_type=jnp.float32) + bh3[...]
    fo[...] = f + rf
    vo[...] = vec[...] + sh
    rv[...] = sh


@functools.cache
def _make_sc_kernels():
    mesh = plsc.VectorSubcoreMesh(core_axis_name="c", subcore_axis_name="s",
                                  num_cores=_NC, num_subcores=_NS)

    @functools.partial(
        pl.kernel,
        out_type=(jax.ShapeDtypeStruct((_E, _H), jnp.float32),
                  jax.ShapeDtypeStruct((_E, 16), jnp.float32)),
        mesh=mesh,
        scratch_types=[
            pltpu.VMEM((_C,), jnp.int32),
            pltpu.VMEM((_C,), jnp.int32),
            pltpu.VMEM((_C, _TROW), jnp.float32),
            pltpu.VMEM((_C, _TROW), jnp.float32),
            pltpu.VMEM((_C, _H), jnp.float32),
            pltpu.VMEM((_C, 16), jnp.float32),
            pltpu.SemaphoreType.DMA,
            pltpu.SemaphoreType.DMA,
        ],
        compiler_params=pltpu.CompilerParams(use_tc_tiling_on_sc=False),
    )
    def _sc_gather(ts_hbm, tr_hbm, s_hbm, r_hbm, g_hbm, cd_hbm,
                   sidx, ridx, buf_s, buf_r, g_v, cd_v, sem_s, sem_r):
        wid = lax.axis_index("s") * _NC + lax.axis_index("c")
        base0 = wid * _EPW

        def chunk(i, carry):
            base = base0 + i * _C
            pltpu.sync_copy(s_hbm.at[pl.ds(base, _C)], sidx)
            pltpu.sync_copy(r_hbm.at[pl.ds(base, _C)], ridx)
            cp_s = pltpu.async_copy(ts_hbm.at[sidx], buf_s, sem_s)
            cp_r = pltpu.async_copy(tr_hbm.at[ridx], buf_r, sem_r)
            cp_s.wait()
            cp_r.wait()

            def row(r, rc):
                for k in range(_H // 16):
                    g_v[r, pl.ds(k * 16, 16)] = (buf_s[r, pl.ds(k * 16, 16)]
                                                 + buf_r[r, pl.ds(k * 16, 16)])
                cd_v[r, pl.ds(0, 16)] = (buf_r[r, pl.ds(_H, 16)]
                                         - buf_s[r, pl.ds(_H, 16)])
                return rc

            lax.fori_loop(0, _C, row, 0)
            pltpu.sync_copy(g_v, g_hbm.at[pl.ds(base, _C), :])
            pltpu.sync_copy(cd_v, cd_hbm.at[pl.ds(base, _C), :])
            return carry

        lax.fori_loop(0, _NCHUNK, chunk, 0)

    @functools.partial(
        pl.kernel,
        out_type=jax.ShapeDtypeStruct((_NC, _N, _OROW), jnp.float32),
        mesh=mesh,
        scratch_types=[
            pltpu.VMEM((_C,), jnp.int32),
            pltpu.VMEM((_C, _OROW), jnp.float32),
            pltpu.VMEM((_ZR, _OROW), jnp.float32),
            pltpu.VMEM_SHARED((_N, _OROW), jnp.float32),
        ],
        compiler_params=pltpu.CompilerParams(use_tc_tiling_on_sc=False),
    )
    def _sc_scatter(mgsh_hbm, r_hbm, out_hbm, ridx, buf, zbuf, accum):
        cid = lax.axis_index("c")
        sid = lax.axis_index("s")
        wid = sid * _NC + cid

        zero16 = jnp.zeros((16,), jnp.float32)
        for r in range(_ZR):
            for k in range(_OROW // 16):
                zbuf[r, pl.ds(k * 16, 16)] = zero16

        def zc(j, carry):
            pltpu.sync_copy(zbuf, accum.at[pl.ds(sid * _NPT + j * _ZR, _ZR), :])
            return carry

        lax.fori_loop(0, _NPT // _ZR, zc, 0)
        plsc.subcore_barrier()

        def chunk(i, carry):
            base = wid * _EPW + i * _C
            pltpu.sync_copy(r_hbm.at[pl.ds(base, _C)], ridx)
            pltpu.sync_copy(mgsh_hbm.at[pl.ds(base, _C), :], buf)
            pltpu.sync_copy(buf, accum.at[ridx], add=True)
            return carry

        lax.fori_loop(0, _NCHUNK, chunk, 0)
        plsc.subcore_barrier()
        pltpu.sync_copy(accum.at[pl.ds(sid * _NPT, _NPT), :],
                        out_hbm.at[cid, pl.ds(sid * _NPT, _NPT), :])

    return _sc_gather, _sc_scatter


def _full(shape):
    return pl.BlockSpec(shape, lambda *_: tuple(0 for _ in shape))


def kernel(node_positions, node_vectors, node_features, senders, receivers,
           We1, be1, We2, be2, Wy1, by1, Wy2, by2, Wyo, byo, Winf, binf,
           Wh1, bh1, Wh2, bh2, Wh3, bh3):
    n, v, _ = node_positions.shape
    pos2 = node_positions.reshape(n, v * 3)
    vec2 = node_vectors.reshape(n, v * 3)

    we1s = We1[:_F]
    we1r = We1[_F:2 * _F]
    wsq = We1[2 * _F:]

    sel = np.zeros((16, 4), np.float32)
    sel[0:3, 0] = 1.0
    sel[3:6, 1] = 1.0
    sel[8:11, 2] = 1.0
    sel[11:14, 3] = 1.0
    sel = jnp.asarray(sel)

    ts, tr = pl.pallas_call(
        _tables_body,
        grid=(_N // _BN,),
        in_specs=[
            pl.BlockSpec((_BN, _F), lambda i: (i, 0)),
            pl.BlockSpec((_BN, 6), lambda i: (i, 0)),
            pl.BlockSpec((_BN, 6), lambda i: (i, 0)),
            _full((_F, _H)),
            _full((_F, _H)),
        ],
        out_specs=[
            pl.BlockSpec((_BN, _TROW), lambda i: (i, 0)),
            pl.BlockSpec((_BN, _TROW), lambda i: (i, 0)),
        ],
        out_shape=[
            jax.ShapeDtypeStruct((_N, _TROW), jnp.float32),
            jax.ShapeDtypeStruct((_N, _TROW), jnp.float32),
        ],
    )(node_features, pos2, vec2, we1s, we1r)

    sc_gather, sc_scatter = _make_sc_kernels()
    if _DEBUG_FAKE_GATHER:
        a = ts[senders]
        b = tr[receivers]
        g = a[:, :_H] + b[:, :_H]
        cd = b[:, _H:_TROW] - a[:, _H:_TROW]
    else:
        g, cd = sc_gather(ts, tr, senders, receivers)

    mgsh = pl.pallas_call(
        _edge_body,
        grid=(_E // _BE,),
        in_specs=[
            pl.BlockSpec((_BE, _H), lambda i: (i, 0)),
            pl.BlockSpec((_BE, 16), lambda i: (i, 0)),
            _full((16, 4)),
            _full((4, _H)),
            _full((1, _H)),
            _full((_H, _H)),
            _full((1, _H)),
            _full((_H, _H)),
            _full((1, _H)),
            _full((_H, _H)),
            _full((1, _H)),
            _full((_H, 2)),
            _full((1, 2)),
            _full((_H, 1)),
            _full((1, 1)),
        ],
        out_specs=pl.BlockSpec((_BE, _OROW), lambda i: (i, 0)),
        out_shape=jax.ShapeDtypeStruct((_E, _OROW), jnp.float32),
    )(g, cd, sel, wsq, be1.reshape(1, _H), We2, be2.reshape(1, _H),
      Wy1, by1.reshape(1, _H), Wy2, by2.reshape(1, _H), Wyo,
      byo.reshape(1, 2), Winf, binf.reshape(1, 1))

    if _DEBUG_FAKE_SCATTER:
        p0 = jax.ops.segment_sum(mgsh, receivers, num_segments=_N)
        partials = jnp.stack([p0, jnp.zeros_like(p0)])
    else:
        partials = sc_scatter(mgsh, receivers)

    fo, vo, rv = pl.pallas_call(
        _node_body,
        grid=(_N // _BN,),
        in_specs=[
            pl.BlockSpec((_NC, _BN, _OROW), lambda i: (0, i, 0)),
            pl.BlockSpec((_BN, _F), lambda i: (i, 0)),
            pl.BlockSpec((_BN, 6), lambda i: (i, 0)),
            _full((_H, _H)),
            _full((_F, _H)),
            _full((1, _H)),
            _full((_H, _H)),
            _full((1, _H)),
            _full((_H, _F)),
            _full((1, _F)),
        ],
        out_specs=[
            pl.BlockSpec((_BN, _F), lambda i: (i, 0)),
            pl.BlockSpec((_BN, 6), lambda i: (i, 0)),
            pl.BlockSpec((_BN, 6), lambda i: (i, 0)),
        ],
        out_shape=[
            jax.ShapeDtypeStruct((_N, _F), jnp.float32),
            jax.ShapeDtypeStruct((_N, 6), jnp.float32),
            jax.ShapeDtypeStruct((_N, 6), jnp.float32),
        ],
    )(partials, node_features, vec2, Wh1[:_H], Wh1[_H:],
      bh1.reshape(1, _H), Wh2, bh2.reshape(1, _H), Wh3, bh3.reshape(1, _F))

    return (node_positions,
            vo.reshape(n, v, 3),
            fo,
            rv.reshape(n, v, 3))


# speedup vs baseline: 33.6325x; 2.1284x over previous
import functools
import math

import jax
import jax.numpy as jnp
import numpy as np
from jax import lax
from jax.experimental import pallas as pl
from jax.experimental.pallas import tpu as pltpu
from jax.experimental.pallas import tpu_sc as plsc

_N = 10000
_E = 320000
_F = 128
_H = 64

_NC = 2
_NS = 16
_NW = _NC * _NS
_EPW = _E // _NW
_C = 80
_NCHUNK = _EPW // _C
_ROW = 128
_NPT = _N // _NS
_ZR = 25

_BN = 1000
_BE = 2000


def _tables_body(feat, pos, vec, we1s, we1r, ts, tr):
    f = feat[...]
    a = jnp.dot(f, we1s[...], preferred_element_type=jnp.float32)
    b = jnp.dot(f, we1r[...], preferred_element_type=jnp.float32)
    z = jnp.zeros((f.shape[0], 2), jnp.float32)
    zp = jnp.zeros((f.shape[0], _ROW - _H - 16), jnp.float32)
    geo = jnp.concatenate([pos[...], z, vec[...], z, zp], axis=1)
    ts[...] = jnp.concatenate([a, geo], axis=1)
    tr[...] = jnp.concatenate([b, geo], axis=1)


def _edge_body(ec, sel, wsq, be1, we2, be2, wy1, by1, wy2, by2, wyo, byo,
               winf, binf, out):
    eb = ec[...]
    gb = eb[:, 0:_H]
    cdb = eb[:, _H:_H + 16]
    d2 = cdb * cdb
    sq = jnp.dot(d2, sel[...], preferred_element_type=jnp.float32)
    pre1 = gb + jnp.dot(sq, wsq[...], preferred_element_type=jnp.float32) + be1[...]
    h1 = jax.nn.silu(pre1)
    m = jax.nn.silu(jnp.dot(h1, we2[...], preferred_element_type=jnp.float32) + be2[...])
    t1 = jax.nn.silu(jnp.dot(m, wy1[...], preferred_element_type=jnp.float32) + by1[...])
    t2 = jax.nn.silu(jnp.dot(t1, wy2[...], preferred_element_type=jnp.float32) + by2[...])
    phi = jnp.dot(t2, wyo[...], preferred_element_type=jnp.float32) + byo[...]
    e = jax.nn.sigmoid(jnp.dot(m, winf[...], preferred_element_type=jnp.float32) + binf[...])
    mg = m * e
    sy0 = sq[:, 2:3]
    sy1 = sq[:, 3:4]
    s0 = phi[:, 0:1] / (1.0 + jnp.sqrt(sy0))
    s1 = phi[:, 1:2] / (1.0 + jnp.sqrt(sy1))
    sh0 = cdb[:, 8:11] * s0
    sh1 = cdb[:, 11:14] * s1
    pad = jnp.zeros((gb.shape[0], _ROW - _H - 6), jnp.float32)
    out[...] = jnp.concatenate([mg, sh0, sh1, pad], axis=1)


def _node_body(p, feat, vec, wh1m, wh1f, bh1, wh2, bh2, wh3, bh3,
               fo, vo, rv):
    inv_avg = 1.0 / float(int(math.sqrt(_E)))
    acc = (p[0] + p[1]) * inv_avg
    m_i = acc[:, 0:_H]
    sh = acc[:, _H:_H + 6]
    f = feat[...]
    h = jax.nn.silu(jnp.dot(m_i, wh1m[...], preferred_element_type=jnp.float32)
                    + jnp.dot(f, wh1f[...], preferred_element_type=jnp.float32)
                    + bh1[...])
    h2 = jax.nn.silu(jnp.dot(h, wh2[...], preferred_element_type=jnp.float32) + bh2[...])
    rf = jnp.dot(h2, wh3[...], preferred_element_type=jnp.float32) + bh3[...]
    fo[...] = f + rf
    vo[...] = vec[...] + sh
    rv[...] = sh


@functools.cache
def _make_sc_kernels():
    mesh = plsc.VectorSubcoreMesh(core_axis_name="c", subcore_axis_name="s",
                                  num_cores=_NC, num_subcores=_NS)

    @functools.partial(
        pl.kernel,
        out_type=jax.ShapeDtypeStruct((_E, _ROW), jnp.float32),
        mesh=mesh,
        scratch_types=[
            pltpu.VMEM((_EPW,), jnp.int32),
            pltpu.VMEM((_EPW,), jnp.int32),
            pltpu.VMEM((2, _C, _ROW), jnp.float32),
            pltpu.VMEM((2, _C, _ROW), jnp.float32),
            pltpu.VMEM((2, _C, _ROW), jnp.float32),
            pltpu.SemaphoreType.DMA,
            pltpu.SemaphoreType.DMA,
            pltpu.SemaphoreType.DMA,
            pltpu.SemaphoreType.DMA,
            pltpu.SemaphoreType.DMA,
            pltpu.SemaphoreType.DMA,
        ],
        compiler_params=pltpu.CompilerParams(use_tc_tiling_on_sc=False),
    )
    def _sc_gather(ts_hbm, tr_hbm, s_hbm, r_hbm, ec_hbm,
                   sidx, ridx, buf_s, buf_r, obuf,
                   sem_s0, sem_s1, sem_r0, sem_r1, sem_w0, sem_w1):
        wid = lax.axis_index("s") * _NC + lax.axis_index("c")
        sem_s = (sem_s0, sem_s1)
        sem_r = (sem_r0, sem_r1)
        sem_w = (sem_w0, sem_w1)

        base0 = wid * _EPW
        pltpu.sync_copy(s_hbm.at[pl.ds(base0, _EPW)], sidx)
        pltpu.sync_copy(r_hbm.at[pl.ds(base0, _EPW)], ridx)

        def fire(i, b):
            sl = pl.ds(i * _C, _C)
            pltpu.make_async_copy(ts_hbm.at[sidx.at[sl]], buf_s.at[b],
                                  sem_s[b]).start()
            pltpu.make_async_copy(tr_hbm.at[ridx.at[sl]], buf_r.at[b],
                                  sem_r[b]).start()

        def wait_g(b):
            sl = pl.ds(0, _C)
            pltpu.make_async_copy(ts_hbm.at[sidx.at[sl]], buf_s.at[b],
                                  sem_s[b]).wait()
            pltpu.make_async_copy(tr_hbm.at[ridx.at[sl]], buf_r.at[b],
                                  sem_r[b]).wait()

        def compute(b):
            def row(r, rc):
                for k in range(_H // 16):
                    sl = pl.ds(k * 16, 16)
                    obuf[b, r, sl] = buf_s[b, r, sl] + buf_r[b, r, sl]
                sl = pl.ds(_H, 16)
                obuf[b, r, sl] = buf_r[b, r, sl] - buf_s[b, r, sl]
                return rc
            lax.fori_loop(0, _C, row, 0)

        def start_write(i, b):
            base = wid * _EPW + i * _C
            pltpu.make_async_copy(obuf.at[b],
                                  ec_hbm.at[pl.ds(base, _C), :],
                                  sem_w[b]).start()

        def wait_w(b):
            pltpu.make_async_copy(obuf.at[b],
                                  ec_hbm.at[pl.ds(0, _C), :],
                                  sem_w[b]).wait()

        fire(0, 0)

        def pair(j, carry):
            i0 = 2 * j

            @pl.when(j > 0)
            def _():
                wait_w(1)
            fire(i0 + 1, 1)
            wait_g(0)

            @pl.when(j > 0)
            def _():
                wait_w(0)
            compute(0)
            start_write(i0, 0)
            fire(i0 + 2, 0)
            wait_g(1)
            compute(1)
            start_write(i0 + 1, 1)
            return carry

        lax.fori_loop(0, (_NCHUNK - 1) // 2, pair, 0)

        wait_g(0)
        wait_w(0)
        compute(0)
        start_write(_NCHUNK - 1, 0)
        wait_w(1)
        wait_w(0)

    @functools.partial(
        pl.kernel,
        out_type=jax.ShapeDtypeStruct((_NC, _N, _ROW), jnp.float32),
        mesh=mesh,
        scratch_types=[
            pltpu.VMEM((2, _C), jnp.int32),
            pltpu.VMEM((2, _C, _ROW), jnp.float32),
            pltpu.VMEM((_ZR, _ROW), jnp.float32),
            pltpu.VMEM_SHARED((_N, _ROW), jnp.float32),
            pltpu.SemaphoreType.DMA,
            pltpu.SemaphoreType.DMA,
            pltpu.SemaphoreType.DMA,
            pltpu.SemaphoreType.DMA,
        ],
        compiler_params=pltpu.CompilerParams(use_tc_tiling_on_sc=False),
    )
    def _sc_scatter(mgsh_hbm, r_hbm, out_hbm, ridx, buf, zbuf, accum,
                    sem_l0, sem_l1, sem_i0, sem_i1):
        cid = lax.axis_index("c")
        sid = lax.axis_index("s")
        wid = sid * _NC + cid
        sem_l = (sem_l0, sem_l1)

        zero16 = jnp.zeros((16,), jnp.float32)
        for r in range(_ZR):
            for k in range(_ROW // 16):
                zbuf[r, pl.ds(k * 16, 16)] = zero16

        def zc(j, carry):
            pltpu.sync_copy(zbuf, accum.at[pl.ds(sid * _NPT + j * _ZR, _ZR), :])
            return carry

        lax.fori_loop(0, _NPT // _ZR, zc, 0)
        plsc.subcore_barrier()

        sem_i = (sem_i0, sem_i1)

        def fire(i, b):
            base = wid * _EPW + i * _C
            pltpu.make_async_copy(mgsh_hbm.at[pl.ds(base, _C), :],
                                  buf.at[b], sem_l[b]).start()
            pltpu.make_async_copy(r_hbm.at[pl.ds(base, _C)],
                                  ridx.at[b], sem_i[b]).start()

        def wait_l(b):
            pltpu.make_async_copy(mgsh_hbm.at[pl.ds(0, _C), :],
                                  buf.at[b], sem_l[b]).wait()
            pltpu.make_async_copy(r_hbm.at[pl.ds(0, _C)],
                                  ridx.at[b], sem_i[b]).wait()

        fire(0, 0)

        def pair(j, carry):
            i0 = 2 * j
            fire(i0 + 1, 1)
            wait_l(0)
            pltpu.sync_copy(buf.at[0], accum.at[ridx.at[0]], add=True)
            fire(i0 + 2, 0)
            wait_l(1)
            pltpu.sync_copy(buf.at[1], accum.at[ridx.at[1]], add=True)
            return carry

        lax.fori_loop(0, (_NCHUNK - 1) // 2, pair, 0)
        wait_l(0)
        pltpu.sync_copy(buf.at[0], accum.at[ridx.at[0]], add=True)

        plsc.subcore_barrier()
        pltpu.sync_copy(accum.at[pl.ds(sid * _NPT, _NPT), :],
                        out_hbm.at[cid, pl.ds(sid * _NPT, _NPT), :])

    return _sc_gather, _sc_scatter


def _full(shape):
    return pl.BlockSpec(shape, lambda *_: tuple(0 for _ in shape))


def kernel(node_positions, node_vectors, node_features, senders, receivers,
           We1, be1, We2, be2, Wy1, by1, Wy2, by2, Wyo, byo, Winf, binf,
           Wh1, bh1, Wh2, bh2, Wh3, bh3):
    n, v, _ = node_positions.shape
    pos2 = node_positions.reshape(n, v * 3)
    vec2 = node_vectors.reshape(n, v * 3)

    we1s = We1[:_F]
    we1r = We1[_F:2 * _F]
    wsq = We1[2 * _F:]

    sel = np.zeros((16, 4), np.float32)
    sel[0:3, 0] = 1.0
    sel[3:6, 1] = 1.0
    sel[8:11, 2] = 1.0
    sel[11:14, 3] = 1.0
    sel = jnp.asarray(sel)

    ts, tr = pl.pallas_call(
        _tables_body,
        grid=(_N // _BN,),
        in_specs=[
            pl.BlockSpec((_BN, _F), lambda i: (i, 0)),
            pl.BlockSpec((_BN, 6), lambda i: (i, 0)),
            pl.BlockSpec((_BN, 6), lambda i: (i, 0)),
            _full((_F, _H)),
            _full((_F, _H)),
        ],
        out_specs=[
            pl.BlockSpec((_BN, _ROW), lambda i: (i, 0)),
            pl.BlockSpec((_BN, _ROW), lambda i: (i, 0)),
        ],
        out_shape=[
            jax.ShapeDtypeStruct((_N, _ROW), jnp.float32),
            jax.ShapeDtypeStruct((_N, _ROW), jnp.float32),
        ],
    )(node_features, pos2, vec2, we1s, we1r)

    sc_gather, sc_scatter = _make_sc_kernels()
    ec = sc_gather(ts, tr, senders, receivers)

    mgsh = pl.pallas_call(
        _edge_body,
        grid=(_E // _BE,),
        in_specs=[
            pl.BlockSpec((_BE, _ROW), lambda i: (i, 0)),
            _full((16, 4)),
            _full((4, _H)),
            _full((1, _H)),
            _full((_H, _H)),
            _full((1, _H)),
            _full((_H, _H)),
            _full((1, _H)),
            _full((_H, _H)),
            _full((1, _H)),
            _full((_H, 2)),
            _full((1, 2)),
            _full((_H, 1)),
            _full((1, 1)),
        ],
        out_specs=pl.BlockSpec((_BE, _ROW), lambda i: (i, 0)),
        out_shape=jax.ShapeDtypeStruct((_E, _ROW), jnp.float32),
    )(ec, sel, wsq, be1.reshape(1, _H), We2, be2.reshape(1, _H),
      Wy1, by1.reshape(1, _H), Wy2, by2.reshape(1, _H), Wyo,
      byo.reshape(1, 2), Winf, binf.reshape(1, 1))

    partials = sc_scatter(mgsh, receivers)

    fo, vo, rv = pl.pallas_call(
        _node_body,
        grid=(_N // _BN,),
        in_specs=[
            pl.BlockSpec((_NC, _BN, _ROW), lambda i: (0, i, 0)),
            pl.BlockSpec((_BN, _F), lambda i: (i, 0)),
            pl.BlockSpec((_BN, 6), lambda i: (i, 0)),
            _full((_H, _H)),
            _full((_F, _H)),
            _full((1, _H)),
            _full((_H, _H)),
            _full((1, _H)),
            _full((_H, _F)),
            _full((1, _F)),
        ],
        out_specs=[
            pl.BlockSpec((_BN, _F), lambda i: (i, 0)),
            pl.BlockSpec((_BN, 6), lambda i: (i, 0)),
            pl.BlockSpec((_BN, 6), lambda i: (i, 0)),
        ],
        out_shape=[
            jax.ShapeDtypeStruct((_N, _F), jnp.float32),
            jax.ShapeDtypeStruct((_N, 6), jnp.float32),
            jax.ShapeDtypeStruct((_N, 6), jnp.float32),
        ],
    )(partials, node_features, vec2, Wh1[:_H], Wh1[_H:],
      bh1.reshape(1, _H), Wh2, bh2.reshape(1, _H), Wh3, bh3.reshape(1, _F))

    return (node_positions,
            vo.reshape(n, v, 3),
            fo,
            rv.reshape(n, v, 3))
